# baseline (speedup 1.0000x reference)
import functools
import jax
import jax.numpy as jnp
from jax import lax
from jax.experimental import pallas as pl
from jax.experimental.pallas import tpu as pltpu
from jax.experimental.pallas import tpu_sc as plsc

_D = 768
_SEQ = 8192
_BATCH = 4
_S_SC = 1024
_S_TC = _SEQ - _S_SC
_BS = 1024
_NW = 32
_C = _S_SC // _NW
_CW = _C * _D


def _tc_add(x, pe_table):
    def body(x_ref, pe_ref, o_ref):
        o_ref[...] = x_ref[...] + pe_ref[...][None, :, :]

    return pl.pallas_call(
        body,
        grid=(_S_TC // _BS,),
        in_specs=[
            pl.BlockSpec((_BATCH, _BS, _D), lambda i: (0, i, 0)),
            pl.BlockSpec((_BS, _D), lambda i: (i, 0)),
        ],
        out_specs=pl.BlockSpec((_BATCH, _BS, _D), lambda i: (0, i, 0)),
        out_shape=jax.ShapeDtypeStruct(x.shape, x.dtype),
    )(x, pe_table)


def _sc_add_tail(xf, pef):
    mesh = plsc.VectorSubcoreMesh(core_axis_name="c", subcore_axis_name="s")

    @functools.partial(
        pl.kernel,
        mesh=mesh,
        out_type=jax.ShapeDtypeStruct((_BATCH * _S_SC * _D,), jnp.float32),
        scratch_types=[
            pltpu.VMEM((2, _CW), jnp.float32),
            pltpu.VMEM((_CW,), jnp.float32),
            pltpu.SemaphoreType.DMA,
            pltpu.SemaphoreType.DMA,
            pltpu.SemaphoreType.DMA,
            pltpu.SemaphoreType.DMA,
            pltpu.SemaphoreType.DMA,
        ],
    )
    def body(x_hbm, pe_hbm, out_hbm, xbuf, pebuf, in0, in1, o0, o1, pe0):
        wid = lax.axis_index("s") * 2 + lax.axis_index("c")
        pos = _S_TC + wid * _C
        in_sem = (in0, in1)
        out_sem = (o0, o1)

        def in_copy(b):
            return pltpu.make_async_copy(
                x_hbm.at[pl.ds((b * _SEQ + pos) * _D, _CW)], xbuf.at[b % 2],
                in_sem[b % 2])

        def out_copy(b):
            return pltpu.make_async_copy(
                xbuf.at[b % 2],
                out_hbm.at[pl.ds((b * _S_SC + wid * _C) * _D, _CW)],
                out_sem[b % 2])

        pe_cp = pltpu.make_async_copy(
            pe_hbm.at[pl.ds(pos * _D, _CW)], pebuf, pe0)
        pe_cp.start()
        in_copy(0).start()

        for b in range(_BATCH):
            if b + 1 < _BATCH:
                if b >= 1:
                    out_copy(b - 1).wait()
                in_copy(b + 1).start()
            in_copy(b).wait()
            if b == 0:
                pe_cp.wait()

            def vadd(j, _):
                sl = pl.ds(j * 16, 16)
                xbuf[b % 2, sl] = xbuf[b % 2, sl] + pebuf[sl]
                return 0

            lax.fori_loop(0, _CW // 16, vadd, 0, unroll=8)
            out_copy(b).start()

        out_copy(_BATCH - 2).wait()
        out_copy(_BATCH - 1).wait()

    return body(xf, pef)


def kernel(x, pe_table):
    xf = jnp.reshape(x, (-1,))
    pef = jnp.reshape(pe_table, (-1,))
    sc_part = jnp.reshape(_sc_add_tail(xf, pef), (_BATCH, _S_SC, _D))
    tc_out = _tc_add(x, pe_table)
    return lax.dynamic_update_slice(tc_out, sc_part, (0, _S_TC, 0))


# speedup vs baseline: 3.3017x; 3.3017x over previous
import jax
import jax.numpy as jnp
from jax.experimental import pallas as pl


_BS = 512


def _add_body(x_ref, pe_ref, o_ref):
    o_ref[...] = x_ref[...] + pe_ref[...][None, :, :]


def kernel(x, pe_table):
    batch, seq, d = x.shape
    num_blocks = seq // _BS
    return pl.pallas_call(
        _add_body,
        grid=(num_blocks,),
        in_specs=[
            pl.BlockSpec((batch, _BS, d), lambda i: (0, i, 0)),
            pl.BlockSpec((_BS, d), lambda i: (i, 0)),
        ],
        out_specs=pl.BlockSpec((batch, _BS, d), lambda i: (0, i, 0)),
        out_shape=jax.ShapeDtypeStruct(x.shape, x.dtype),
    )(x, pe_table)
